# baseline (speedup 1.0000x reference)
import functools

import jax
import jax.numpy as jnp
from jax import lax
from jax.experimental import pallas as pl
from jax.experimental.pallas import tpu as pltpu
from jax.experimental.pallas import tpu_sc as plsc

_B, _S, _D = 4, 4096, 2048
_N = _B * _S
_NC, _NS = 2, 16
_NW = _NC * _NS
_SPW = _S // _NW
_CS = 4
_RPC = _CS * _B
_NCS = _SPW // _CS
_NPH = 3
_VPU = _RPC * _D // 16

_mesh = plsc.VectorSubcoreMesh(core_axis_name="c", subcore_axis_name="s")


@functools.partial(
    pl.kernel,
    out_type=jax.ShapeDtypeStruct((_N, _D), jnp.float32),
    mesh=_mesh,
    scratch_types=[
        pltpu.VMEM((_NCS * _RPC,), jnp.int32),
        pltpu.VMEM((2, _CS, _D), jnp.float32),
        pltpu.VMEM((_NPH, _RPC, _D), jnp.float32),
        pltpu.SemaphoreType.DMA,
        pltpu.SemaphoreType.DMA,
        pltpu.SemaphoreType.DMA,
        pltpu.SemaphoreType.DMA,
        pltpu.SemaphoreType.DMA,
        pltpu.SemaphoreType.DMA,
        pltpu.SemaphoreType.DMA,
    ],
)
def _embed_sc(ids_hbm, tok_hbm, pos_hbm, out_hbm, midx_v, pos_v, res_v,
              sem_pos, sg0, sg1, sg2, so0, so1, so2):
    gsems = (sg0, sg1, sg2)
    osems = (so0, so1, so2)
    wid = lax.axis_index("s") * _NC + lax.axis_index("c")
    s0 = wid * _SPW

    def issue_pos(cs, p):
        pltpu.async_copy(pos_hbm.at[pl.ds(s0 + cs * _CS, _CS)], pos_v.at[p],
                         sem_pos)

    def issue_gather(cs, ph):
        pltpu.async_copy(tok_hbm.at[midx_v.at[pl.ds(cs * _RPC, _RPC)]],
                         res_v.at[ph], gsems[ph])

    def wait_gather(ph):
        pltpu.make_async_copy(pos_hbm.at[pl.ds(0, _RPC)], res_v.at[ph],
                              gsems[ph]).wait()

    def wait_outs(ph):
        for _ in range(_B):
            pltpu.make_async_copy(pos_hbm.at[pl.ds(0, _CS)],
                                  res_v.at[ph].at[pl.ds(0, _CS)],
                                  osems[ph]).wait()

    pltpu.async_copy(ids_hbm.at[wid], midx_v, osems[0])
    issue_pos(0, 0)
    pltpu.make_async_copy(ids_hbm.at[0], midx_v, osems[0]).wait()

    issue_gather(0, 0)
    issue_gather(1, 1)

    @pl.loop(0, _NCS - 2, step=6)
    def _chunks(j):
        for q in range(6):
            cs = j + q
            ph = q % _NPH
            rph = (q + 2) % _NPH
            p = q & 1
            pltpu.make_async_copy(pos_hbm.at[pl.ds(0, _CS)], pos_v.at[p],
                                  sem_pos).wait()
            issue_pos(cs + 1, 1 - p)

            wait_gather(ph)
            res_c = res_v.at[ph]
            pos_p = pos_v.at[p]

            @plsc.parallel_loop(0, _VPU, unroll=8)
            def _add(t):
                i = lax.shift_right_logical(t, 7)
                pr = lax.bitwise_and(i, _CS - 1)
                k = pl.multiple_of(
                    lax.shift_left(lax.bitwise_and(t, 127), 4), 16)
                plsc.addupdate(res_c.at[i, pl.ds(k, 16)],
                               pos_p[pr, pl.ds(k, 16)])

            for b in range(_B):
                pltpu.async_copy(
                    res_c.at[pl.ds(b * _CS, _CS)],
                    out_hbm.at[pl.ds(b * _S + s0 + cs * _CS, _CS)],
                    osems[ph])

            @pl.when(cs >= 1)
            def _():
                wait_outs(rph)

            issue_gather(cs + 2, rph)

    for cs, ph, rph, p in ((_NCS - 2, 0, 2, 0), (_NCS - 1, 1, 0, 1)):
        pltpu.make_async_copy(pos_hbm.at[pl.ds(0, _CS)], pos_v.at[p],
                              sem_pos).wait()
        if cs + 1 < _NCS:
            issue_pos(cs + 1, 1 - p)
        wait_gather(ph)
        res_c = res_v.at[ph]
        pos_p = pos_v.at[p]

        @plsc.parallel_loop(0, _VPU, unroll=8)
        def _add(t):
            i = lax.shift_right_logical(t, 7)
            pr = lax.bitwise_and(i, _CS - 1)
            k = pl.multiple_of(lax.shift_left(lax.bitwise_and(t, 127), 4), 16)
            plsc.addupdate(res_c.at[i, pl.ds(k, 16)], pos_p[pr, pl.ds(k, 16)])

        for b in range(_B):
            pltpu.async_copy(
                res_c.at[pl.ds(b * _CS, _CS)],
                out_hbm.at[pl.ds(b * _S + s0 + cs * _CS, _CS)],
                osems[ph])
        wait_outs(rph)

    wait_outs(1)


def kernel(token_ids, tok_table, pos_table):
    ids_cm = (token_ids.astype(jnp.int32)
              .reshape(_B, _NW, _NCS, _CS)
              .transpose(1, 2, 0, 3)
              .reshape(_NW, _NCS * _RPC))
    out = _embed_sc(ids_cm, tok_table, pos_table)
    return out.reshape(_B, _S, _D)


# speedup vs baseline: 1.0634x; 1.0634x over previous
import functools

import jax
import jax.numpy as jnp
from jax import lax
from jax.experimental import pallas as pl
from jax.experimental.pallas import tpu as pltpu
from jax.experimental.pallas import tpu_sc as plsc

_B, _S, _D = 4, 4096, 2048
_N = _B * _S
_NC, _NS = 2, 16
_NW = _NC * _NS
_SPW = _S // _NW
_CS = 4
_NCS = _SPW // _CS
_NSL = 2 * _B
_VPU = _CS * _D // 16

_mesh = plsc.VectorSubcoreMesh(core_axis_name="c", subcore_axis_name="s")


@functools.partial(
    pl.kernel,
    out_type=jax.ShapeDtypeStruct((_N, _D), jnp.float32),
    mesh=_mesh,
    scratch_types=[
        pltpu.VMEM((_B, _SPW), jnp.int32),
        pltpu.VMEM((2, _CS, _D), jnp.float32),
        pltpu.VMEM((_NSL, _CS, _D), jnp.float32),
        pltpu.SemaphoreType.DMA,
    ] + [pltpu.SemaphoreType.DMA] * _NSL
      + [pltpu.SemaphoreType.DMA] * _NSL,
)
def _embed_sc(ids_hbm, tok_hbm, pos_hbm, out_hbm, idx_v, pos_v, res_v,
              sem_pos, *sems):
    gsems = sems[:_NSL]
    osems = sems[_NSL:]
    wid = lax.axis_index("s") * _NC + lax.axis_index("c")
    s0 = wid * _SPW

    def issue_pos(cs, p):
        pltpu.async_copy(pos_hbm.at[pl.ds(s0 + cs * _CS, _CS)], pos_v.at[p],
                         sem_pos)

    def issue_gather(cs, b, sl):
        pltpu.async_copy(tok_hbm.at[idx_v.at[b, pl.ds(cs * _CS, _CS)]],
                         res_v.at[sl], gsems[sl])

    def wait_gather(sl):
        pltpu.make_async_copy(pos_hbm.at[pl.ds(0, _CS)], res_v.at[sl],
                              gsems[sl]).wait()

    def wait_out(sl):
        pltpu.make_async_copy(pos_hbm.at[pl.ds(0, _CS)], res_v.at[sl],
                              osems[sl]).wait()

    for b in range(_B):
        pltpu.async_copy(ids_hbm.at[b, pl.ds(s0, _SPW)], idx_v.at[b],
                         osems[b])
    issue_pos(0, 0)
    for b in range(_B):
        pltpu.make_async_copy(ids_hbm.at[0, pl.ds(0, _SPW)], idx_v.at[b],
                              osems[b]).wait()
    for b in range(_B):
        issue_gather(0, b, b)

    @pl.loop(0, _NCS, step=2)
    def _chunks(j):
        for p in (0, 1):
            cs = j + p
            pltpu.make_async_copy(pos_hbm.at[pl.ds(0, _CS)], pos_v.at[p],
                                  sem_pos).wait()

            @pl.when(cs + 1 < _NCS)
            def _():
                issue_pos(cs + 1, 1 - p)

            pos_p = pos_v.at[p]
            for b in range(_B):
                sl = p * _B + b
                osl = (1 - p) * _B + b
                wait_gather(sl)
                res_b = res_v.at[sl]

                @plsc.parallel_loop(0, _VPU, unroll=16)
                def _add(t):
                    i = lax.shift_right_logical(t, 7)
                    k = pl.multiple_of(
                        lax.shift_left(lax.bitwise_and(t, 127), 4), 16)
                    plsc.addupdate(res_b.at[i, pl.ds(k, 16)],
                                   pos_p[i, pl.ds(k, 16)])

                pltpu.async_copy(
                    res_b,
                    out_hbm.at[pl.ds(b * _S + s0 + cs * _CS, _CS)],
                    osems[sl])

                @pl.when(cs >= 1)
                def _():
                    wait_out(osl)

                @pl.when(cs + 1 < _NCS)
                def _():
                    issue_gather(cs + 1, b, osl)

    for b in range(_B):
        wait_out(((_NCS - 1) % 2) * _B + b)


def kernel(token_ids, tok_table, pos_table):
    ids = token_ids.astype(jnp.int32)
    out = _embed_sc(ids, tok_table, pos_table)
    return out.reshape(_B, _S, _D)


# speedup vs baseline: 1.0692x; 1.0055x over previous
import functools

import jax
import jax.numpy as jnp
from jax import lax
from jax.experimental import pallas as pl
from jax.experimental.pallas import tpu as pltpu
from jax.experimental.pallas import tpu_sc as plsc

_B, _S, _D = 4, 4096, 2048
_N = _B * _S
_NC, _NS = 2, 16
_NW = _NC * _NS
_SPW = _S // _NW
_CS = 4
_NCS = _SPW // _CS
_NSL = 2 * _B
_VPU = _CS * _D // 16

_mesh = plsc.VectorSubcoreMesh(core_axis_name="c", subcore_axis_name="s")


@functools.partial(
    pl.kernel,
    out_type=jax.ShapeDtypeStruct((_N, _D), jnp.float32),
    mesh=_mesh,
    scratch_types=[
        pltpu.VMEM((_B, _SPW), jnp.int32),
        pltpu.VMEM((2, _CS, _D), jnp.float32),
        pltpu.VMEM((_NSL, _CS, _D), jnp.float32),
        pltpu.SemaphoreType.DMA,
    ] + [pltpu.SemaphoreType.DMA] * _NSL
      + [pltpu.SemaphoreType.DMA] * _NSL,
)
def _embed_sc(ids_hbm, tok_hbm, pos_hbm, out_hbm, idx_v, pos_v, res_v,
              sem_pos, *sems):
    gsems = sems[:_NSL]
    osems = sems[_NSL:]
    wid = lax.axis_index("s") * _NC + lax.axis_index("c")
    s0 = wid * _SPW

    def issue_pos(cs, p):
        pltpu.async_copy(pos_hbm.at[pl.ds(s0 + cs * _CS, _CS)], pos_v.at[p],
                         sem_pos)

    def issue_gather(cs, b, sl):
        pltpu.async_copy(tok_hbm.at[idx_v.at[b, pl.ds(cs * _CS, _CS)]],
                         res_v.at[sl], gsems[sl])

    def wait_gather(sl):
        pltpu.make_async_copy(pos_hbm.at[pl.ds(0, _CS)], res_v.at[sl],
                              gsems[sl]).wait()

    def wait_out(sl):
        pltpu.make_async_copy(pos_hbm.at[pl.ds(0, _CS)], res_v.at[sl],
                              osems[sl]).wait()

    for b in range(_B):
        pltpu.async_copy(ids_hbm.at[b, pl.ds(s0, _SPW)], idx_v.at[b],
                         osems[b])
    issue_pos(0, 0)
    for b in range(_B):
        pltpu.make_async_copy(ids_hbm.at[0, pl.ds(0, _SPW)], idx_v.at[b],
                              osems[b]).wait()
    for b in range(_B):
        issue_gather(0, b, b)

    @pl.loop(0, _NCS, step=2)
    def _chunks(j):
        for p in (0, 1):
            cs = j + p
            pltpu.make_async_copy(pos_hbm.at[pl.ds(0, _CS)], pos_v.at[p],
                                  sem_pos).wait()

            @pl.when(cs + 1 < _NCS)
            def _():
                issue_pos(cs + 1, 1 - p)

            pos_p = pos_v.at[p]
            for b in range(_B):
                sl = p * _B + b
                osl = (1 - p) * _B + b
                wait_gather(sl)
                res_b = res_v.at[sl]

                @plsc.parallel_loop(0, _VPU, unroll=8)
                def _add(t):
                    i = lax.shift_right_logical(t, 7)
                    k = pl.multiple_of(
                        lax.shift_left(lax.bitwise_and(t, 127), 4), 16)
                    plsc.addupdate(res_b.at[i, pl.ds(k, 16)],
                                   pos_p[i, pl.ds(k, 16)])

                pltpu.async_copy(
                    res_b,
                    out_hbm.at[pl.ds(b * _S + s0 + cs * _CS, _CS)],
                    osems[sl])

                @pl.when(cs >= 1)
                def _():
                    wait_out(osl)

                @pl.when(cs + 1 < _NCS)
                def _():
                    issue_gather(cs + 1, b, osl)

    for b in range(_B):
        wait_out(((_NCS - 1) % 2) * _B + b)


def kernel(token_ids, tok_table, pos_table):
    ids = token_ids.astype(jnp.int32)
    out = _embed_sc(ids, tok_table, pos_table)
    return out.reshape(_B, _S, _D)
